# baseline (speedup 1.0000x reference)
import functools

import jax
import jax.numpy as jnp
from jax import lax
from jax.experimental import pallas as pl
from jax.experimental.pallas import tpu as pltpu
from jax.experimental.pallas import tpu_sc as plsc

N = 10000
N_PAD = 10240
E = 320000
F_IN = 128
H = 256
C = 40
C_PAD = 64

NTILE = 16
HALF = N_PAD // 2
ACC_ROWS = HALF + 8
TROWS = HALF // NTILE
CAP = E // NTILE
NCB = 10
NPAIR = NCB // 2

BLK = 2048

_MESH = dict(core_axis_name="c", subcore_axis_name="s",
             num_cores=2, num_subcores=NTILE)


@functools.lru_cache(maxsize=None)
def _make_prepass():
  mesh = plsc.VectorSubcoreMesh(**_MESH)
  out_type = (jax.ShapeDtypeStruct((2, NTILE, CAP), jnp.int32),
              jax.ShapeDtypeStruct((2, NTILE, CAP), jnp.int32),
              jax.ShapeDtypeStruct((2, NTILE, 16), jnp.int32))

  @functools.partial(
      pl.kernel, mesh=mesh, out_type=out_type,
      compiler_params=pltpu.CompilerParams(use_tc_tiling_on_sc=False,
                                           needs_layout_passes=False),
      scratch_types=[
          pltpu.VMEM((CAP,), jnp.int32),
          pltpu.VMEM((CAP,), jnp.int32),
          pltpu.VMEM((CAP + 16,), jnp.int32),
          pltpu.VMEM((CAP + 16,), jnp.int32),
          pltpu.VMEM((16,), jnp.int32),
      ],
  )
  def prepass(src_hbm, dst_hbm, fsrc_hbm, fdst_hbm, cnt_hbm,
              rsrc, rdst, osrc, odst, cntv):
    c = lax.axis_index("c")
    s = lax.axis_index("s")
    lo = c * HALF

    pltpu.sync_copy(src_hbm.at[s], rsrc)
    pltpu.sync_copy(dst_hbm.at[s], rdst)

    dummy_src = jnp.zeros((16,), jnp.int32)
    dummy_dst = jnp.full((16,), HALF, jnp.int32)

    def prefill(i, carry):
      sl = pl.ds(i * 16, 16)
      osrc[sl] = dummy_src
      odst[sl] = dummy_dst
      return carry

    lax.fori_loop(0, CAP // 16 + 1, prefill, 0)

    def compact(i, pos):
      sl = pl.ds(i * 16, 16)
      sv = rsrc[sl]
      loc = rdst[sl] - lo
      m = (loc >= 0) & (loc < HALF)
      plsc.store_compressed(osrc.at[pl.ds(pos, 16)], sv, mask=m)
      plsc.store_compressed(odst.at[pl.ds(pos, 16)], loc, mask=m)
      pc = plsc.all_reduce_population_count(m)
      return pos + pc[0]

    n = lax.fori_loop(0, CAP // 16, compact, jnp.int32(0))

    cntv[...] = jnp.full((16,), 1, jnp.int32) * n
    pltpu.sync_copy(osrc.at[pl.ds(0, CAP)], fsrc_hbm.at[c, s])
    pltpu.sync_copy(odst.at[pl.ds(0, CAP)], fdst_hbm.at[c, s])
    pltpu.sync_copy(cntv, cnt_hbm.at[c, s])

  return prepass


@functools.lru_cache(maxsize=None)
def _make_sc_agg(d, ch):
  mesh = plsc.VectorSubcoreMesh(**_MESH)
  out_type = jax.ShapeDtypeStruct((N_PAD, d), jnp.float32)

  @functools.partial(
      pl.kernel, mesh=mesh, out_type=out_type,
      compiler_params=pltpu.CompilerParams(use_tc_tiling_on_sc=False),
      scratch_types=[
          pltpu.VMEM_SHARED((ACC_ROWS, d), jnp.float32),
          pltpu.VMEM((NCB, ch), jnp.int32),
          pltpu.VMEM((NCB, ch), jnp.int32),
          pltpu.VMEM((ch, d), jnp.float32),
          pltpu.VMEM((ch, d), jnp.float32),
          pltpu.VMEM((16,), jnp.int32),
          pltpu.SemaphoreType.DMA,
          pltpu.SemaphoreType.DMA,
      ],
  )
  def agg_kernel(x_hbm, src_hbm, dst_hbm, cnt_hbm, z_hbm, out_hbm,
                 acc, src_v, dst_v, buf0, buf1, cntv, g0, g1):
    c = lax.axis_index("c")
    s = lax.axis_index("s")
    arow = pl.ds(s * TROWS, TROWS)

    pltpu.sync_copy(cnt_hbm.at[c, s], cntv)
    pltpu.sync_copy(z_hbm, acc.at[arow])
    plsc.subcore_barrier()

    n = cntv[...][0]
    nblk = (n + NCB * ch - 1) // (NCB * ch)

    def outer(b, carry):
      blk = pl.ds(b * NCB, NCB)
      pltpu.sync_copy(src_hbm.at[c, s, blk], src_v)
      pltpu.sync_copy(dst_hbm.at[c, s, blk], dst_v)
      pltpu.async_copy(x_hbm.at[src_v.at[0]], buf0, g0)

      def pair(i, carry2):
        j0 = 2 * i
        pltpu.make_async_copy(x_hbm.at[src_v.at[j0]], buf0, g0).wait()
        pltpu.async_copy(x_hbm.at[src_v.at[j0 + 1]], buf1, g1)
        pltpu.sync_copy(buf0, acc.at[dst_v.at[j0]], add=True)
        pltpu.make_async_copy(x_hbm.at[src_v.at[j0 + 1]], buf1, g1).wait()

        @pl.when(i + 1 < NPAIR)
        def _():
          pltpu.async_copy(x_hbm.at[src_v.at[j0 + 2]], buf0, g0)

        pltpu.sync_copy(buf1, acc.at[dst_v.at[j0 + 1]], add=True)
        return carry2

      lax.fori_loop(0, NPAIR, pair, 0)
      return carry

    lax.fori_loop(0, nblk, outer, 0)
    plsc.subcore_barrier()
    pltpu.sync_copy(acc.at[arow], out_hbm.at[pl.ds(c * HALF + s * TROWS,
                                                   TROWS)])

  return agg_kernel


def _row_spec(d):
  return pl.BlockSpec((BLK, d), lambda i: (i, 0))


def _full_spec(r, c_):
  return pl.BlockSpec((r, c_), lambda i: (0, 0))


def _layer1_body(a_ref, x_ref, wl_ref, wr_ref, b_ref, h_ref, inv_ref):
  cnt = a_ref[:, 128:129]
  inv = 1.0 / jnp.maximum(cnt, 1.0)
  g = jnp.dot(a_ref[:, :128], wl_ref[...], preferred_element_type=jnp.float32)
  h = g * inv + jnp.dot(x_ref[...], wr_ref[...],
                        preferred_element_type=jnp.float32) + b_ref[...]
  h_ref[...] = jnp.maximum(h, 0.0)
  inv_ref[...] = inv


def _layer1_tc(a, x, wl, wr, b):
  return pl.pallas_call(
      _layer1_body,
      grid=(N_PAD // BLK,),
      in_specs=[_row_spec(144), _row_spec(F_IN),
                _full_spec(F_IN, H), _full_spec(F_IN, H), _full_spec(1, H)],
      out_specs=(_row_spec(H), _row_spec(1)),
      out_shape=(jax.ShapeDtypeStruct((N_PAD, H), jnp.float32),
                 jax.ShapeDtypeStruct((N_PAD, 1), jnp.float32)),
  )(a, x, wl, wr, b)


def _layer2_body(a_ref, h_ref, inv_ref, wl_ref, wr_ref, b_ref, o_ref):
  g = jnp.dot(a_ref[...], wl_ref[...], preferred_element_type=jnp.float32)
  r = jnp.dot(h_ref[...], wr_ref[...], preferred_element_type=jnp.float32)
  o_ref[...] = jnp.maximum(g * inv_ref[...] + r + b_ref[...], 0.0)


def _layer2_tc(a, h, inv, wl, wr, b):
  return pl.pallas_call(
      _layer2_body,
      grid=(N_PAD // BLK,),
      in_specs=[_row_spec(H), _row_spec(H), _row_spec(1),
                _full_spec(H, H), _full_spec(H, H), _full_spec(1, H)],
      out_specs=_row_spec(H),
      out_shape=jax.ShapeDtypeStruct((N_PAD, H), jnp.float32),
  )(a, h, inv, wl, wr, b)


def _layer3_body(h_ref, wl_ref, wr_ref, b_ref, p_ref, r_ref):
  p_ref[...] = jnp.dot(h_ref[...], wl_ref[...],
                       preferred_element_type=jnp.float32)
  r_ref[...] = jnp.dot(h_ref[...], wr_ref[...],
                       preferred_element_type=jnp.float32) + b_ref[...]


def _layer3_tc(h, wl, wr, b):
  return pl.pallas_call(
      _layer3_body,
      grid=(N_PAD // BLK,),
      in_specs=[_row_spec(H),
                _full_spec(H, C_PAD), _full_spec(H, C_PAD),
                _full_spec(1, C_PAD)],
      out_specs=(_row_spec(C_PAD), _row_spec(C_PAD)),
      out_shape=(jax.ShapeDtypeStruct((N_PAD, C_PAD), jnp.float32),
                 jax.ShapeDtypeStruct((N_PAD, C_PAD), jnp.float32)),
  )(h, wl, wr, b)


def _final_body(a_ref, r_ref, inv_ref, out_ref):
  v = a_ref[...] * inv_ref[...] + r_ref[...]
  col = lax.broadcasted_iota(jnp.int32, (BLK, C_PAD), 1)
  valid = col < C
  mx = jnp.max(jnp.where(valid, v, -jnp.inf), axis=1, keepdims=True)
  e = jnp.where(valid, jnp.exp(v - mx), 0.0)
  lse = jnp.log(jnp.sum(e, axis=1, keepdims=True))
  out_ref[...] = (v - mx - lse)[:, :C]


def _final_tc(a, r, inv):
  return pl.pallas_call(
      _final_body,
      grid=(N_PAD // BLK,),
      in_specs=[_row_spec(C_PAD), _row_spec(C_PAD), _row_spec(1)],
      out_specs=_row_spec(C),
      out_shape=jax.ShapeDtypeStruct((N_PAD, C), jnp.float32),
  )(a, r, inv)


def kernel(x, edge_index, Wl1, Wr1, b1, Wl2, Wr2, b2, Wl3, Wr3, b3):
  src2 = edge_index[0].astype(jnp.int32).reshape(NTILE, CAP)
  dst2 = edge_index[1].astype(jnp.int32).reshape(NTILE, CAP)
  fsrc, fdst, cnt = _make_prepass()(src2, dst2)

  def idx4(ch):
    nc = CAP // ch
    return fsrc.reshape(2, NTILE, nc, ch), fdst.reshape(2, NTILE, nc, ch)

  xp = jnp.pad(x, ((0, N_PAD - N), (0, 0)))
  xaug = jnp.concatenate([xp, jnp.ones((N_PAD, 16), jnp.float32)], axis=1)

  s4, d4 = idx4(125)
  a1 = _make_sc_agg(144, 125)(xaug, s4, d4, cnt,
                              jnp.zeros((TROWS, 144), jnp.float32))
  h, inv = _layer1_tc(a1, xp, Wl1, Wr1, b1.reshape(1, H))

  s4b, d4b = idx4(80)
  a2 = _make_sc_agg(256, 80)(h, s4b, d4b, cnt,
                             jnp.zeros((TROWS, 256), jnp.float32))
  h2 = _layer2_tc(a2, h, inv, Wl2, Wr2, b2.reshape(1, H))

  wl3 = jnp.pad(Wl3, ((0, 0), (0, C_PAD - C)))
  wr3 = jnp.pad(Wr3, ((0, 0), (0, C_PAD - C)))
  b3p = jnp.pad(b3, (0, C_PAD - C)).reshape(1, C_PAD)
  p, r3 = _layer3_tc(h2, wl3, wr3, b3p)

  a3 = _make_sc_agg(C_PAD, 125)(p, s4, d4, cnt,
                                jnp.zeros((TROWS, C_PAD), jnp.float32))
  return _final_tc(a3, r3, inv)[:N]


# speedup vs baseline: 1.6656x; 1.6656x over previous
import functools

import jax
import jax.numpy as jnp
from jax import lax
from jax.experimental import pallas as pl
from jax.experimental.pallas import tpu as pltpu
from jax.experimental.pallas import tpu_sc as plsc

N = 10000
N_PAD = 10240
E = 320000
F_IN = 128
H = 256
C = 40
C_PAD = 64

NTILE = 16
HALF = N_PAD // 2
ACC_ROWS = HALF + 8
TROWS = HALF // NTILE
CAP = E // NTILE
NCB = 10
NPAIR = NCB // 2

BLK = 2048

_MESH = dict(core_axis_name="c", subcore_axis_name="s",
             num_cores=2, num_subcores=NTILE)


@functools.lru_cache(maxsize=None)
def _make_prepass():
  mesh = plsc.VectorSubcoreMesh(**_MESH)
  out_type = (jax.ShapeDtypeStruct((2, NTILE, CAP), jnp.int32),
              jax.ShapeDtypeStruct((2, NTILE, CAP), jnp.int32),
              jax.ShapeDtypeStruct((2, NTILE, 16), jnp.int32))

  @functools.partial(
      pl.kernel, mesh=mesh, out_type=out_type,
      compiler_params=pltpu.CompilerParams(use_tc_tiling_on_sc=False,
                                           needs_layout_passes=False),
      scratch_types=[
          pltpu.VMEM((CAP,), jnp.int32),
          pltpu.VMEM((CAP,), jnp.int32),
          pltpu.VMEM((CAP + 16,), jnp.int32),
          pltpu.VMEM((CAP + 16,), jnp.int32),
          pltpu.VMEM((16,), jnp.int32),
      ],
  )
  def prepass(src_hbm, dst_hbm, fsrc_hbm, fdst_hbm, cnt_hbm,
              rsrc, rdst, osrc, odst, cntv):
    c = lax.axis_index("c")
    s = lax.axis_index("s")
    lo = c * HALF

    pltpu.sync_copy(src_hbm.at[s], rsrc)
    pltpu.sync_copy(dst_hbm.at[s], rdst)

    dummy_src = jnp.zeros((16,), jnp.int32)
    dummy_dst = jnp.full((16,), HALF, jnp.int32)

    def prefill(i, carry):
      sl = pl.ds(i * 16, 16)
      osrc[sl] = dummy_src
      odst[sl] = dummy_dst
      return carry

    lax.fori_loop(0, CAP // 16 + 1, prefill, 0)

    def compact(i, pos):
      sl = pl.ds(i * 16, 16)
      sv = rsrc[sl]
      loc = rdst[sl] - lo
      m = (loc >= 0) & (loc < HALF)
      plsc.store_compressed(osrc.at[pl.ds(pos, 16)], sv, mask=m)
      plsc.store_compressed(odst.at[pl.ds(pos, 16)], loc, mask=m)
      pc = plsc.all_reduce_population_count(m)
      return pos + pc[0]

    n = lax.fori_loop(0, CAP // 16, compact, jnp.int32(0))

    cntv[...] = jnp.full((16,), 1, jnp.int32) * n
    pltpu.sync_copy(osrc.at[pl.ds(0, CAP)], fsrc_hbm.at[c, s])
    pltpu.sync_copy(odst.at[pl.ds(0, CAP)], fdst_hbm.at[c, s])
    pltpu.sync_copy(cntv, cnt_hbm.at[c, s])

  return prepass


@functools.lru_cache(maxsize=None)
def _make_sc_agg(d, ch):
  mesh = plsc.VectorSubcoreMesh(**_MESH)
  out_type = jax.ShapeDtypeStruct((N_PAD, d), jnp.float32)

  @functools.partial(
      pl.kernel, mesh=mesh, out_type=out_type,
      compiler_params=pltpu.CompilerParams(use_tc_tiling_on_sc=False),
      scratch_types=[
          pltpu.VMEM_SHARED((ACC_ROWS, d), jnp.float32),
          pltpu.VMEM((NCB, ch), jnp.int32),
          pltpu.VMEM((NCB, ch), jnp.int32),
          pltpu.VMEM((ch, d), jnp.float32),
          pltpu.VMEM((ch, d), jnp.float32),
          pltpu.VMEM((16,), jnp.int32),
          pltpu.SemaphoreType.DMA,
          pltpu.SemaphoreType.DMA,
      ],
  )
  def agg_kernel(x_hbm, src_hbm, dst_hbm, cnt_hbm, z_hbm, out_hbm,
                 acc, src_v, dst_v, buf0, buf1, cntv, g0, g1):
    c = lax.axis_index("c")
    s = lax.axis_index("s")
    arow = pl.ds(s * TROWS, TROWS)

    pltpu.sync_copy(cnt_hbm.at[c, s], cntv)
    pltpu.sync_copy(z_hbm, acc.at[arow])
    plsc.subcore_barrier()

    n = cntv[...][0]
    del n
    nblk = -(-(CAP // 2) // (NCB * ch))

    def outer(b, carry):
      blk = pl.ds(b * NCB, NCB)
      pltpu.sync_copy(src_hbm.at[c, s, blk], src_v)
      pltpu.sync_copy(dst_hbm.at[c, s, blk], dst_v)
      pltpu.async_copy(x_hbm.at[src_v.at[0]], buf0, g0)

      def pair(i, carry2):
        j0 = 2 * i
        pltpu.make_async_copy(x_hbm.at[src_v.at[j0]], buf0, g0).wait()
        pltpu.async_copy(x_hbm.at[src_v.at[j0 + 1]], buf1, g1)
        pltpu.sync_copy(buf0, acc.at[dst_v.at[j0]], add=True)
        pltpu.make_async_copy(x_hbm.at[src_v.at[j0 + 1]], buf1, g1).wait()

        @pl.when(i + 1 < NPAIR)
        def _():
          pltpu.async_copy(x_hbm.at[src_v.at[j0 + 2]], buf0, g0)

        pltpu.sync_copy(buf1, acc.at[dst_v.at[j0 + 1]], add=True)
        return carry2

      lax.fori_loop(0, NPAIR, pair, 0)
      return carry

    lax.fori_loop(0, nblk, outer, 0)
    plsc.subcore_barrier()
    pltpu.sync_copy(acc.at[arow], out_hbm.at[pl.ds(c * HALF + s * TROWS,
                                                   TROWS)])

  return agg_kernel


def _row_spec(d):
  return pl.BlockSpec((BLK, d), lambda i: (i, 0))


def _full_spec(r, c_):
  return pl.BlockSpec((r, c_), lambda i: (0, 0))


def _layer1_body(a_ref, x_ref, wl_ref, wr_ref, b_ref, h_ref, inv_ref):
  cnt = a_ref[:, 128:129]
  inv = 1.0 / jnp.maximum(cnt, 1.0)
  g = jnp.dot(a_ref[:, :128], wl_ref[...], preferred_element_type=jnp.float32)
  h = g * inv + jnp.dot(x_ref[...], wr_ref[...],
                        preferred_element_type=jnp.float32) + b_ref[...]
  h_ref[...] = jnp.maximum(h, 0.0)
  inv_ref[...] = inv


def _layer1_tc(a, x, wl, wr, b):
  return pl.pallas_call(
      _layer1_body,
      grid=(N_PAD // BLK,),
      in_specs=[_row_spec(144), _row_spec(F_IN),
                _full_spec(F_IN, H), _full_spec(F_IN, H), _full_spec(1, H)],
      out_specs=(_row_spec(H), _row_spec(1)),
      out_shape=(jax.ShapeDtypeStruct((N_PAD, H), jnp.float32),
                 jax.ShapeDtypeStruct((N_PAD, 1), jnp.float32)),
  )(a, x, wl, wr, b)


def _layer2_body(a_ref, h_ref, inv_ref, wl_ref, wr_ref, b_ref, o_ref):
  g = jnp.dot(a_ref[...], wl_ref[...], preferred_element_type=jnp.float32)
  r = jnp.dot(h_ref[...], wr_ref[...], preferred_element_type=jnp.float32)
  o_ref[...] = jnp.maximum(g * inv_ref[...] + r + b_ref[...], 0.0)


def _layer2_tc(a, h, inv, wl, wr, b):
  return pl.pallas_call(
      _layer2_body,
      grid=(N_PAD // BLK,),
      in_specs=[_row_spec(H), _row_spec(H), _row_spec(1),
                _full_spec(H, H), _full_spec(H, H), _full_spec(1, H)],
      out_specs=_row_spec(H),
      out_shape=jax.ShapeDtypeStruct((N_PAD, H), jnp.float32),
  )(a, h, inv, wl, wr, b)


def _layer3_body(h_ref, wl_ref, wr_ref, b_ref, p_ref, r_ref):
  p_ref[...] = jnp.dot(h_ref[...], wl_ref[...],
                       preferred_element_type=jnp.float32)
  r_ref[...] = jnp.dot(h_ref[...], wr_ref[...],
                       preferred_element_type=jnp.float32) + b_ref[...]


def _layer3_tc(h, wl, wr, b):
  return pl.pallas_call(
      _layer3_body,
      grid=(N_PAD // BLK,),
      in_specs=[_row_spec(H),
                _full_spec(H, C_PAD), _full_spec(H, C_PAD),
                _full_spec(1, C_PAD)],
      out_specs=(_row_spec(C_PAD), _row_spec(C_PAD)),
      out_shape=(jax.ShapeDtypeStruct((N_PAD, C_PAD), jnp.float32),
                 jax.ShapeDtypeStruct((N_PAD, C_PAD), jnp.float32)),
  )(h, wl, wr, b)


def _final_body(a_ref, r_ref, inv_ref, out_ref):
  v = a_ref[...] * inv_ref[...] + r_ref[...]
  col = lax.broadcasted_iota(jnp.int32, (BLK, C_PAD), 1)
  valid = col < C
  mx = jnp.max(jnp.where(valid, v, -jnp.inf), axis=1, keepdims=True)
  e = jnp.where(valid, jnp.exp(v - mx), 0.0)
  lse = jnp.log(jnp.sum(e, axis=1, keepdims=True))
  out_ref[...] = (v - mx - lse)[:, :C]


def _final_tc(a, r, inv):
  return pl.pallas_call(
      _final_body,
      grid=(N_PAD // BLK,),
      in_specs=[_row_spec(C_PAD), _row_spec(C_PAD), _row_spec(1)],
      out_specs=_row_spec(C),
      out_shape=jax.ShapeDtypeStruct((N_PAD, C), jnp.float32),
  )(a, r, inv)


def kernel(x, edge_index, Wl1, Wr1, b1, Wl2, Wr2, b2, Wl3, Wr3, b3):
  src2 = edge_index[0].astype(jnp.int32).reshape(NTILE, CAP)
  dst2 = edge_index[1].astype(jnp.int32).reshape(NTILE, CAP)
  fsrc, fdst, cnt = _make_prepass()(src2, dst2)

  def idx4(ch):
    nc = CAP // ch
    return fsrc.reshape(2, NTILE, nc, ch), fdst.reshape(2, NTILE, nc, ch)

  xp = jnp.pad(x, ((0, N_PAD - N), (0, 0)))
  xaug = jnp.concatenate([xp, jnp.ones((N_PAD, 16), jnp.float32)], axis=1)

  s4, d4 = idx4(125)
  a1 = _make_sc_agg(144, 125)(xaug, s4, d4, cnt,
                              jnp.zeros((TROWS, 144), jnp.float32))
  h, inv = _layer1_tc(a1, xp, Wl1, Wr1, b1.reshape(1, H))

  s4b, d4b = idx4(80)
  a2 = _make_sc_agg(256, 80)(h, s4b, d4b, cnt,
                             jnp.zeros((TROWS, 256), jnp.float32))
  h2 = _layer2_tc(a2, h, inv, Wl2, Wr2, b2.reshape(1, H))

  wl3 = jnp.pad(Wl3, ((0, 0), (0, C_PAD - C)))
  wr3 = jnp.pad(Wr3, ((0, 0), (0, C_PAD - C)))
  b3p = jnp.pad(b3, (0, C_PAD - C)).reshape(1, C_PAD)
  p, r3 = _layer3_tc(h2, wl3, wr3, b3p)

  a3 = _make_sc_agg(C_PAD, 125)(p, s4, d4, cnt,
                                jnp.zeros((TROWS, C_PAD), jnp.float32))
  return _final_tc(a3, r3, inv)[:N]


# speedup vs baseline: 2.9629x; 1.7788x over previous
import functools

import jax
import jax.numpy as jnp
from jax import lax
from jax.experimental import pallas as pl
from jax.experimental.pallas import tpu as pltpu
from jax.experimental.pallas import tpu_sc as plsc

N = 10000
N_PAD = 10240
E = 320000
F_IN = 128
H = 256
C = 40
C_PAD = 64

NTILE = 16
ROWS_PER_TILE = N_PAD // NTILE
EDGES_PER_TILE = E // NTILE
CHUNK = 125
NCHUNK = EDGES_PER_TILE // CHUNK
NCB = 10
NBLK = NCHUNK // NCB
NPAIR = NCB // 2

BLK = 2048


@functools.lru_cache(maxsize=None)
def _make_sc_agg(d):
  mesh = plsc.VectorSubcoreMesh(core_axis_name="c", subcore_axis_name="s",
                                num_cores=2, num_subcores=NTILE)
  out_type = (jax.ShapeDtypeStruct((N_PAD, d), jnp.float32),
              jax.ShapeDtypeStruct((N_PAD, d), jnp.float32))

  @functools.partial(
      pl.kernel, mesh=mesh, out_type=out_type,
      compiler_params=pltpu.CompilerParams(use_tc_tiling_on_sc=False),
      scratch_types=[
          pltpu.VMEM_SHARED((N_PAD, d), jnp.float32),
          pltpu.VMEM((NCB, CHUNK), jnp.int32),
          pltpu.VMEM((NCB, CHUNK), jnp.int32),
          pltpu.VMEM((CHUNK, d), jnp.float32),
          pltpu.VMEM((CHUNK, d), jnp.float32),
          pltpu.SemaphoreType.DMA,
          pltpu.SemaphoreType.DMA,
          pltpu.SemaphoreType.DMA,
          pltpu.SemaphoreType.DMA,
      ],
  )
  def agg_kernel(x0_hbm, x1_hbm, src_hbm, dst_hbm, z_hbm,
                 out0_hbm, out1_hbm, acc, src_v, dst_v, buf0, buf1,
                 g0, g1, s0, s1):
    c = lax.axis_index("c")
    s = lax.axis_index("s")
    rows = pl.ds(s * ROWS_PER_TILE, ROWS_PER_TILE)

    pltpu.sync_copy(z_hbm, acc.at[rows])
    plsc.subcore_barrier()

    def run(x_hbm, out_hbm):
      def outer(b, carry):
        blk = pl.ds(b * NCB, NCB)
        pltpu.sync_copy(src_hbm.at[s, blk], src_v)
        pltpu.sync_copy(dst_hbm.at[s, blk], dst_v)
        pltpu.async_copy(x_hbm.at[src_v.at[0]], buf0, g0)

        def pair(i, carry2):
          j0 = 2 * i
          pltpu.make_async_copy(x_hbm.at[src_v.at[j0]], buf0, g0).wait()
          pltpu.async_copy(buf0, acc.at[dst_v.at[j0]], s0, add=True)
          pltpu.make_async_copy(x_hbm.at[src_v.at[j0 + 1]], buf1, g1).wait()
          pltpu.async_copy(buf1, acc.at[dst_v.at[j0 + 1]], s1, add=True)
          pltpu.make_async_copy(buf0, acc.at[dst_v.at[j0]], s0).wait()

          @pl.when(i + 1 < NPAIR)
          def _():
            pltpu.async_copy(x_hbm.at[src_v.at[j0 + 2]], buf0, g0)

          pltpu.make_async_copy(buf1, acc.at[dst_v.at[j0 + 1]], s1).wait()

          @pl.when(i + 1 < NPAIR)
          def _():
            pltpu.async_copy(x_hbm.at[src_v.at[j0 + 3]], buf1, g1)

          return carry2

        pltpu.async_copy(x_hbm.at[src_v.at[1]], buf1, g1)
        lax.fori_loop(0, NPAIR, pair, 0)
        return carry

      lax.fori_loop(0, NBLK, outer, 0)
      plsc.subcore_barrier()
      pltpu.sync_copy(acc.at[rows], out_hbm.at[rows])

    @pl.when(c == 0)
    def _():
      run(x0_hbm, out0_hbm)

    @pl.when(c == 1)
    def _():
      run(x1_hbm, out1_hbm)

  return agg_kernel


def _agg80(*args):
  return _make_sc_agg(80)(*args)


def _agg128(*args):
  return _make_sc_agg(128)(*args)


def _agg32(*args):
  return _make_sc_agg(C_PAD // 2)(*args)


def _row_spec(d):
  return pl.BlockSpec((BLK, d), lambda i: (i, 0))


def _full_spec(r, c_):
  return pl.BlockSpec((r, c_), lambda i: (0, 0))


def _layer1_body(a0_ref, a1_ref, x_ref, wla_ref, wlb_ref, wr_ref, b_ref,
                 h0_ref, h1_ref, inv_ref):
  cnt = a0_ref[:, 64:65]
  inv = 1.0 / jnp.maximum(cnt, 1.0)
  g = jnp.dot(a0_ref[:, :64], wla_ref[...], preferred_element_type=jnp.float32)
  g += jnp.dot(a1_ref[:, :64], wlb_ref[...], preferred_element_type=jnp.float32)
  h = g * inv + jnp.dot(x_ref[...], wr_ref[...],
                        preferred_element_type=jnp.float32) + b_ref[...]
  h = jnp.maximum(h, 0.0)
  h0_ref[...] = h[:, :128]
  h1_ref[...] = h[:, 128:]
  inv_ref[...] = inv


def _layer1_tc(a0, a1, x, wla, wlb, wr, b):
  return pl.pallas_call(
      _layer1_body,
      grid=(N_PAD // BLK,),
      in_specs=[_row_spec(80), _row_spec(80), _row_spec(F_IN),
                _full_spec(64, H), _full_spec(64, H), _full_spec(F_IN, H),
                _full_spec(1, H)],
      out_specs=(_row_spec(128), _row_spec(128), _row_spec(1)),
      out_shape=(jax.ShapeDtypeStruct((N_PAD, 128), jnp.float32),
                 jax.ShapeDtypeStruct((N_PAD, 128), jnp.float32),
                 jax.ShapeDtypeStruct((N_PAD, 1), jnp.float32)),
  )(a0, a1, x, wla, wlb, wr, b)


def _layer2_body(a0_ref, a1_ref, h0_ref, h1_ref, inv_ref,
                 wla_ref, wlb_ref, wra_ref, wrb_ref, b_ref,
                 o0_ref, o1_ref):
  g = jnp.dot(a0_ref[...], wla_ref[...], preferred_element_type=jnp.float32)
  g += jnp.dot(a1_ref[...], wlb_ref[...], preferred_element_type=jnp.float32)
  r = jnp.dot(h0_ref[...], wra_ref[...], preferred_element_type=jnp.float32)
  r += jnp.dot(h1_ref[...], wrb_ref[...], preferred_element_type=jnp.float32)
  h = g * inv_ref[...] + r + b_ref[...]
  h = jnp.maximum(h, 0.0)
  o0_ref[...] = h[:, :128]
  o1_ref[...] = h[:, 128:]


def _layer2_tc(a0, a1, h0, h1, inv, wla, wlb, wra, wrb, b):
  return pl.pallas_call(
      _layer2_body,
      grid=(N_PAD // BLK,),
      in_specs=[_row_spec(128), _row_spec(128), _row_spec(128), _row_spec(128),
                _row_spec(1),
                _full_spec(128, H), _full_spec(128, H),
                _full_spec(128, H), _full_spec(128, H), _full_spec(1, H)],
      out_specs=(_row_spec(128), _row_spec(128)),
      out_shape=(jax.ShapeDtypeStruct((N_PAD, 128), jnp.float32),
                 jax.ShapeDtypeStruct((N_PAD, 128), jnp.float32)),
  )(a0, a1, h0, h1, inv, wla, wlb, wra, wrb, b)


def _layer3_body(h0_ref, h1_ref, wla_ref, wlb_ref, wra_ref, wrb_ref, b_ref,
                 p0_ref, p1_ref, r_ref):
  p = jnp.dot(h0_ref[...], wla_ref[...], preferred_element_type=jnp.float32)
  p += jnp.dot(h1_ref[...], wlb_ref[...], preferred_element_type=jnp.float32)
  r = jnp.dot(h0_ref[...], wra_ref[...], preferred_element_type=jnp.float32)
  r += jnp.dot(h1_ref[...], wrb_ref[...], preferred_element_type=jnp.float32)
  p0_ref[...] = p[:, :C_PAD // 2]
  p1_ref[...] = p[:, C_PAD // 2:]
  r_ref[...] = r + b_ref[...]


def _layer3_tc(h0, h1, wla, wlb, wra, wrb, b):
  half = C_PAD // 2
  return pl.pallas_call(
      _layer3_body,
      grid=(N_PAD // BLK,),
      in_specs=[_row_spec(128), _row_spec(128),
                _full_spec(128, C_PAD), _full_spec(128, C_PAD),
                _full_spec(128, C_PAD), _full_spec(128, C_PAD),
                _full_spec(1, C_PAD)],
      out_specs=(_row_spec(half), _row_spec(half), _row_spec(C_PAD)),
      out_shape=(jax.ShapeDtypeStruct((N_PAD, half), jnp.float32),
                 jax.ShapeDtypeStruct((N_PAD, half), jnp.float32),
                 jax.ShapeDtypeStruct((N_PAD, C_PAD), jnp.float32)),
  )(h0, h1, wla, wlb, wra, wrb, b)


def _final_body(a0_ref, a1_ref, r_ref, inv_ref, out_ref):
  v = jnp.concatenate([a0_ref[...], a1_ref[...]], axis=1) * inv_ref[...]
  v = v + r_ref[...]
  col = lax.broadcasted_iota(jnp.int32, (BLK, C_PAD), 1)
  valid = col < C
  mx = jnp.max(jnp.where(valid, v, -jnp.inf), axis=1, keepdims=True)
  e = jnp.where(valid, jnp.exp(v - mx), 0.0)
  lse = jnp.log(jnp.sum(e, axis=1, keepdims=True))
  out_ref[...] = (v - mx - lse)[:, :C]


def _final_tc(a0, a1, r, inv):
  half = C_PAD // 2
  return pl.pallas_call(
      _final_body,
      grid=(N_PAD // BLK,),
      in_specs=[_row_spec(half), _row_spec(half), _row_spec(C_PAD),
                _row_spec(1)],
      out_specs=_row_spec(C),
      out_shape=jax.ShapeDtypeStruct((N_PAD, C), jnp.float32),
  )(a0, a1, r, inv)


def kernel(x, edge_index, Wl1, Wr1, b1, Wl2, Wr2, b2, Wl3, Wr3, b3):
  src = edge_index[0].astype(jnp.int32)
  dst = edge_index[1].astype(jnp.int32)
  src3 = src.reshape(NTILE, NCHUNK, CHUNK)
  dst3 = dst.reshape(NTILE, NCHUNK, CHUNK)

  xp = jnp.pad(x, ((0, N_PAD - N), (0, 0)))
  ones = jnp.ones((N_PAD, 16), jnp.float32)
  x0 = jnp.concatenate([xp[:, :64], ones], axis=1)
  x1 = jnp.concatenate([xp[:, 64:], ones], axis=1)
  z80 = jnp.zeros((ROWS_PER_TILE, 80), jnp.float32)
  a10, a11 = _agg80(x0, x1, src3, dst3, z80)

  h0, h1, inv = _layer1_tc(a10, a11, xp, Wl1[:64], Wl1[64:], Wr1,
                           b1.reshape(1, H))

  z128 = jnp.zeros((ROWS_PER_TILE, 128), jnp.float32)
  a20, a21 = _agg128(h0, h1, src3, dst3, z128)

  h20, h21 = _layer2_tc(a20, a21, h0, h1, inv, Wl2[:128], Wl2[128:],
                        Wr2[:128], Wr2[128:], b2.reshape(1, H))

  wl3 = jnp.pad(Wl3, ((0, 0), (0, C_PAD - C)))
  wr3 = jnp.pad(Wr3, ((0, 0), (0, C_PAD - C)))
  b3p = jnp.pad(b3, (0, C_PAD - C)).reshape(1, C_PAD)
  p0, p1, r3 = _layer3_tc(h20, h21, wl3[:128], wl3[128:],
                          wr3[:128], wr3[128:], b3p)

  z32 = jnp.zeros((ROWS_PER_TILE, C_PAD // 2), jnp.float32)
  a30, a31 = _agg32(p0, p1, src3, dst3, z32)

  return _final_tc(a30, a31, r3, inv)[:N]


# speedup vs baseline: 3.1081x; 1.0490x over previous
import functools

import jax
import jax.numpy as jnp
from jax import lax
from jax.experimental import pallas as pl
from jax.experimental.pallas import tpu as pltpu
from jax.experimental.pallas import tpu_sc as plsc

N = 10000
N_PAD = 10240
E = 320000
F_IN = 128
H = 256
C = 40
C_PAD = 64

NTILE = 16
ROWS_PER_TILE = N_PAD // NTILE
EDGES_PER_TILE = E // NTILE
CHUNK = 125
NCHUNK = EDGES_PER_TILE // CHUNK
NCB = 20
NBLK = NCHUNK // NCB
NPAIR = NCB // 2

BLK = 2048


@functools.lru_cache(maxsize=None)
def _make_sc_agg(d):
  mesh = plsc.VectorSubcoreMesh(core_axis_name="c", subcore_axis_name="s",
                                num_cores=2, num_subcores=NTILE)
  out_type = (jax.ShapeDtypeStruct((N_PAD, d), jnp.float32),
              jax.ShapeDtypeStruct((N_PAD, d), jnp.float32))

  @functools.partial(
      pl.kernel, mesh=mesh, out_type=out_type,
      compiler_params=pltpu.CompilerParams(use_tc_tiling_on_sc=False),
      scratch_types=[
          pltpu.VMEM_SHARED((N_PAD, d), jnp.float32),
          pltpu.VMEM((NCB, CHUNK), jnp.int32),
          pltpu.VMEM((NCB, CHUNK), jnp.int32),
          pltpu.VMEM((CHUNK, d), jnp.float32),
          pltpu.VMEM((CHUNK, d), jnp.float32),
          pltpu.SemaphoreType.DMA,
          pltpu.SemaphoreType.DMA,
          pltpu.SemaphoreType.DMA,
          pltpu.SemaphoreType.DMA,
      ],
  )
  def agg_kernel(x0_hbm, x1_hbm, src_hbm, dst_hbm, z_hbm,
                 out0_hbm, out1_hbm, acc, src_v, dst_v, buf0, buf1,
                 g0, g1, s0, s1):
    c = lax.axis_index("c")
    s = lax.axis_index("s")
    rows = pl.ds(s * ROWS_PER_TILE, ROWS_PER_TILE)

    pltpu.sync_copy(z_hbm, acc.at[rows])
    plsc.subcore_barrier()

    def run(x_hbm, out_hbm):
      def outer(b, carry):
        blk = pl.ds(b * NCB, NCB)
        pltpu.sync_copy(src_hbm.at[s, blk], src_v)
        pltpu.sync_copy(dst_hbm.at[s, blk], dst_v)
        pltpu.async_copy(x_hbm.at[src_v.at[0]], buf0, g0)

        def pair(i, carry2):
          j0 = 2 * i
          pltpu.make_async_copy(x_hbm.at[src_v.at[j0]], buf0, g0).wait()
          pltpu.async_copy(buf0, acc.at[dst_v.at[j0]], s0, add=True)
          pltpu.make_async_copy(x_hbm.at[src_v.at[j0 + 1]], buf1, g1).wait()
          pltpu.async_copy(buf1, acc.at[dst_v.at[j0 + 1]], s1, add=True)
          pltpu.make_async_copy(buf0, acc.at[dst_v.at[j0]], s0).wait()

          @pl.when(i + 1 < NPAIR)
          def _():
            pltpu.async_copy(x_hbm.at[src_v.at[j0 + 2]], buf0, g0)

          pltpu.make_async_copy(buf1, acc.at[dst_v.at[j0 + 1]], s1).wait()

          @pl.when(i + 1 < NPAIR)
          def _():
            pltpu.async_copy(x_hbm.at[src_v.at[j0 + 3]], buf1, g1)

          return carry2

        pltpu.async_copy(x_hbm.at[src_v.at[1]], buf1, g1)
        lax.fori_loop(0, NPAIR, pair, 0)
        return carry

      lax.fori_loop(0, NBLK, outer, 0)
      plsc.subcore_barrier()
      pltpu.sync_copy(acc.at[rows], out_hbm.at[rows])

    @pl.when(c == 0)
    def _():
      run(x0_hbm, out0_hbm)

    @pl.when(c == 1)
    def _():
      run(x1_hbm, out1_hbm)

  return agg_kernel


def _agg80(*args):
  return _make_sc_agg(80)(*args)


def _agg128(*args):
  return _make_sc_agg(128)(*args)


def _agg32(*args):
  return _make_sc_agg(C_PAD // 2)(*args)


def _row_spec(d):
  return pl.BlockSpec((BLK, d), lambda i: (i, 0))


def _full_spec(r, c_):
  return pl.BlockSpec((r, c_), lambda i: (0, 0))


def _layer1_body(a0_ref, a1_ref, x_ref, wla_ref, wlb_ref, wr_ref, b_ref,
                 h0_ref, h1_ref, inv_ref):
  cnt = a0_ref[:, 64:65]
  inv = 1.0 / jnp.maximum(cnt, 1.0)
  g = jnp.dot(a0_ref[:, :64], wla_ref[...], preferred_element_type=jnp.float32)
  g += jnp.dot(a1_ref[:, :64], wlb_ref[...], preferred_element_type=jnp.float32)
  h = g * inv + jnp.dot(x_ref[...], wr_ref[...],
                        preferred_element_type=jnp.float32) + b_ref[...]
  h = jnp.maximum(h, 0.0)
  h0_ref[...] = h[:, :128]
  h1_ref[...] = h[:, 128:]
  inv_ref[...] = inv


def _layer1_tc(a0, a1, x, wla, wlb, wr, b):
  return pl.pallas_call(
      _layer1_body,
      grid=(N_PAD // BLK,),
      in_specs=[_row_spec(80), _row_spec(80), _row_spec(F_IN),
                _full_spec(64, H), _full_spec(64, H), _full_spec(F_IN, H),
                _full_spec(1, H)],
      out_specs=(_row_spec(128), _row_spec(128), _row_spec(1)),
      out_shape=(jax.ShapeDtypeStruct((N_PAD, 128), jnp.float32),
                 jax.ShapeDtypeStruct((N_PAD, 128), jnp.float32),
                 jax.ShapeDtypeStruct((N_PAD, 1), jnp.float32)),
  )(a0, a1, x, wla, wlb, wr, b)


def _layer23_body(a0_ref, a1_ref, h0_ref, h1_ref, inv_ref,
                  wla_ref, wlb_ref, wra_ref, wrb_ref, b_ref,
                  wl3_ref, wr3_ref, b3_ref,
                  p0_ref, p1_ref, r3_ref):
  g = jnp.dot(a0_ref[...], wla_ref[...], preferred_element_type=jnp.float32)
  g += jnp.dot(a1_ref[...], wlb_ref[...], preferred_element_type=jnp.float32)
  r = jnp.dot(h0_ref[...], wra_ref[...], preferred_element_type=jnp.float32)
  r += jnp.dot(h1_ref[...], wrb_ref[...], preferred_element_type=jnp.float32)
  h2 = jnp.maximum(g * inv_ref[...] + r + b_ref[...], 0.0)
  p = jnp.dot(h2, wl3_ref[...], preferred_element_type=jnp.float32)
  p0_ref[...] = p[:, :C_PAD // 2]
  p1_ref[...] = p[:, C_PAD // 2:]
  r3_ref[...] = jnp.dot(h2, wr3_ref[...],
                        preferred_element_type=jnp.float32) + b3_ref[...]


def _layer23_tc(a0, a1, h0, h1, inv, wla, wlb, wra, wrb, b, wl3, wr3, b3):
  half = C_PAD // 2
  return pl.pallas_call(
      _layer23_body,
      grid=(N_PAD // BLK,),
      in_specs=[_row_spec(128), _row_spec(128), _row_spec(128), _row_spec(128),
                _row_spec(1),
                _full_spec(128, H), _full_spec(128, H),
                _full_spec(128, H), _full_spec(128, H), _full_spec(1, H),
                _full_spec(H, C_PAD), _full_spec(H, C_PAD),
                _full_spec(1, C_PAD)],
      out_specs=(_row_spec(half), _row_spec(half), _row_spec(C_PAD)),
      out_shape=(jax.ShapeDtypeStruct((N_PAD, half), jnp.float32),
                 jax.ShapeDtypeStruct((N_PAD, half), jnp.float32),
                 jax.ShapeDtypeStruct((N_PAD, C_PAD), jnp.float32)),
  )(a0, a1, h0, h1, inv, wla, wlb, wra, wrb, b, wl3, wr3, b3)


def _final_body(a0_ref, a1_ref, r_ref, inv_ref, out_ref):
  v = jnp.concatenate([a0_ref[...], a1_ref[...]], axis=1) * inv_ref[...]
  v = v + r_ref[...]
  col = lax.broadcasted_iota(jnp.int32, (BLK, C_PAD), 1)
  valid = col < C
  mx = jnp.max(jnp.where(valid, v, -jnp.inf), axis=1, keepdims=True)
  e = jnp.where(valid, jnp.exp(v - mx), 0.0)
  lse = jnp.log(jnp.sum(e, axis=1, keepdims=True))
  out_ref[...] = (v - mx - lse)[:, :C]


def _final_tc(a0, a1, r, inv):
  half = C_PAD // 2
  return pl.pallas_call(
      _final_body,
      grid=(N_PAD // BLK,),
      in_specs=[_row_spec(half), _row_spec(half), _row_spec(C_PAD),
                _row_spec(1)],
      out_specs=_row_spec(C),
      out_shape=jax.ShapeDtypeStruct((N_PAD, C), jnp.float32),
  )(a0, a1, r, inv)


def kernel(x, edge_index, Wl1, Wr1, b1, Wl2, Wr2, b2, Wl3, Wr3, b3):
  src = edge_index[0].astype(jnp.int32)
  dst = edge_index[1].astype(jnp.int32)
  src3 = src.reshape(NTILE, NCHUNK, CHUNK)
  dst3 = dst.reshape(NTILE, NCHUNK, CHUNK)

  xp = jnp.pad(x, ((0, N_PAD - N), (0, 0)))
  ones = jnp.ones((N_PAD, 16), jnp.float32)
  x0 = jnp.concatenate([xp[:, :64], ones], axis=1)
  x1 = jnp.concatenate([xp[:, 64:], ones], axis=1)
  z80 = jnp.zeros((ROWS_PER_TILE, 80), jnp.float32)
  a10, a11 = _agg80(x0, x1, src3, dst3, z80)

  h0, h1, inv = _layer1_tc(a10, a11, xp, Wl1[:64], Wl1[64:], Wr1,
                           b1.reshape(1, H))

  z128 = jnp.zeros((ROWS_PER_TILE, 128), jnp.float32)
  a20, a21 = _agg128(h0, h1, src3, dst3, z128)

  wl3 = jnp.pad(Wl3, ((0, 0), (0, C_PAD - C)))
  wr3 = jnp.pad(Wr3, ((0, 0), (0, C_PAD - C)))
  b3p = jnp.pad(b3, (0, C_PAD - C)).reshape(1, C_PAD)
  p0, p1, r3 = _layer23_tc(a20, a21, h0, h1, inv, Wl2[:128], Wl2[128:],
                           Wr2[:128], Wr2[128:], b2.reshape(1, H),
                           wl3, wr3, b3p)

  z32 = jnp.zeros((ROWS_PER_TILE, C_PAD // 2), jnp.float32)
  a30, a31 = _agg32(p0, p1, src3, dst3, z32)

  return _final_tc(a30, a31, r3, inv)[:N]


# speedup vs baseline: 3.1799x; 1.0231x over previous
import functools

import jax
import jax.numpy as jnp
from jax import lax
from jax.experimental import pallas as pl
from jax.experimental.pallas import tpu as pltpu
from jax.experimental.pallas import tpu_sc as plsc

N = 10000
N_PAD = 10240
E = 320000
F_IN = 128
H = 256
C = 40
C_PAD = 64

NTILE = 16
ROWS_PER_TILE = N_PAD // NTILE
EDGES_PER_TILE = E // NTILE
CHUNK = 125
NCHUNK = EDGES_PER_TILE // CHUNK
NCB = 40
NBLK = NCHUNK // NCB
NPAIR = NCB // 2

BLK = 2048


@functools.lru_cache(maxsize=None)
def _make_sc_agg(d):
  mesh = plsc.VectorSubcoreMesh(core_axis_name="c", subcore_axis_name="s",
                                num_cores=2, num_subcores=NTILE)
  out_type = (jax.ShapeDtypeStruct((N_PAD, d), jnp.float32),
              jax.ShapeDtypeStruct((N_PAD, d), jnp.float32))

  @functools.partial(
      pl.kernel, mesh=mesh, out_type=out_type,
      compiler_params=pltpu.CompilerParams(use_tc_tiling_on_sc=False),
      scratch_types=[
          pltpu.VMEM_SHARED((N_PAD, d), jnp.float32),
          pltpu.VMEM((NCB, CHUNK), jnp.int32),
          pltpu.VMEM((NCB, CHUNK), jnp.int32),
          pltpu.VMEM((CHUNK, d), jnp.float32),
          pltpu.VMEM((CHUNK, d), jnp.float32),
          pltpu.SemaphoreType.DMA,
          pltpu.SemaphoreType.DMA,
          pltpu.SemaphoreType.DMA,
          pltpu.SemaphoreType.DMA,
      ],
  )
  def agg_kernel(x0_hbm, x1_hbm, src_hbm, dst_hbm, z_hbm,
                 out0_hbm, out1_hbm, acc, src_v, dst_v, buf0, buf1,
                 g0, g1, s0, s1):
    c = lax.axis_index("c")
    s = lax.axis_index("s")
    rows = pl.ds(s * ROWS_PER_TILE, ROWS_PER_TILE)

    pltpu.sync_copy(z_hbm, acc.at[rows])
    plsc.subcore_barrier()

    def run(x_hbm, out_hbm):
      def outer(b, carry):
        blk = pl.ds(b * NCB, NCB)
        pltpu.sync_copy(src_hbm.at[s, blk], src_v)
        pltpu.sync_copy(dst_hbm.at[s, blk], dst_v)
        pltpu.async_copy(x_hbm.at[src_v.at[0]], buf0, g0)

        def pair(i, carry2):
          j0 = 2 * i
          pltpu.make_async_copy(x_hbm.at[src_v.at[j0]], buf0, g0).wait()
          pltpu.async_copy(buf0, acc.at[dst_v.at[j0]], s0, add=True)
          pltpu.make_async_copy(x_hbm.at[src_v.at[j0 + 1]], buf1, g1).wait()
          pltpu.async_copy(buf1, acc.at[dst_v.at[j0 + 1]], s1, add=True)
          pltpu.make_async_copy(buf0, acc.at[dst_v.at[j0]], s0).wait()

          @pl.when(i + 1 < NPAIR)
          def _():
            pltpu.async_copy(x_hbm.at[src_v.at[j0 + 2]], buf0, g0)

          pltpu.make_async_copy(buf1, acc.at[dst_v.at[j0 + 1]], s1).wait()

          @pl.when(i + 1 < NPAIR)
          def _():
            pltpu.async_copy(x_hbm.at[src_v.at[j0 + 3]], buf1, g1)

          return carry2

        pltpu.async_copy(x_hbm.at[src_v.at[1]], buf1, g1)
        lax.fori_loop(0, NPAIR, pair, 0)
        return carry

      lax.fori_loop(0, NBLK, outer, 0)
      plsc.subcore_barrier()
      pltpu.sync_copy(acc.at[rows], out_hbm.at[rows])

    @pl.when(c == 0)
    def _():
      run(x0_hbm, out0_hbm)

    @pl.when(c == 1)
    def _():
      run(x1_hbm, out1_hbm)

  return agg_kernel


def _agg80(*args):
  return _make_sc_agg(80)(*args)


def _agg128(*args):
  return _make_sc_agg(128)(*args)


def _agg32(*args):
  return _make_sc_agg(C_PAD // 2)(*args)


def _row_spec(d):
  return pl.BlockSpec((BLK, d), lambda i: (i, 0))


def _full_spec(r, c_):
  return pl.BlockSpec((r, c_), lambda i: (0, 0))


def _layer1_body(a0_ref, a1_ref, x_ref, wla_ref, wlb_ref, wr_ref, b_ref,
                 h0_ref, h1_ref, inv_ref):
  cnt = a0_ref[:, 64:65]
  inv = 1.0 / jnp.maximum(cnt, 1.0)
  g = jnp.dot(a0_ref[:, :64], wla_ref[...], preferred_element_type=jnp.float32)
  g += jnp.dot(a1_ref[:, :64], wlb_ref[...], preferred_element_type=jnp.float32)
  h = g * inv + jnp.dot(x_ref[...], wr_ref[...],
                        preferred_element_type=jnp.float32) + b_ref[...]
  h = jnp.maximum(h, 0.0)
  h0_ref[...] = h[:, :128]
  h1_ref[...] = h[:, 128:]
  inv_ref[...] = inv


def _layer1_tc(a0, a1, x, wla, wlb, wr, b):
  return pl.pallas_call(
      _layer1_body,
      grid=(N_PAD // BLK,),
      in_specs=[_row_spec(80), _row_spec(80), _row_spec(F_IN),
                _full_spec(64, H), _full_spec(64, H), _full_spec(F_IN, H),
                _full_spec(1, H)],
      out_specs=(_row_spec(128), _row_spec(128), _row_spec(1)),
      out_shape=(jax.ShapeDtypeStruct((N_PAD, 128), jnp.float32),
                 jax.ShapeDtypeStruct((N_PAD, 128), jnp.float32),
                 jax.ShapeDtypeStruct((N_PAD, 1), jnp.float32)),
  )(a0, a1, x, wla, wlb, wr, b)


def _layer23_body(a0_ref, a1_ref, h0_ref, h1_ref, inv_ref,
                  wla_ref, wlb_ref, wra_ref, wrb_ref, b_ref,
                  wl3_ref, wr3_ref, b3_ref,
                  p0_ref, p1_ref, r3_ref):
  g = jnp.dot(a0_ref[...], wla_ref[...], preferred_element_type=jnp.float32)
  g += jnp.dot(a1_ref[...], wlb_ref[...], preferred_element_type=jnp.float32)
  r = jnp.dot(h0_ref[...], wra_ref[...], preferred_element_type=jnp.float32)
  r += jnp.dot(h1_ref[...], wrb_ref[...], preferred_element_type=jnp.float32)
  h2 = jnp.maximum(g * inv_ref[...] + r + b_ref[...], 0.0)
  p = jnp.dot(h2, wl3_ref[...], preferred_element_type=jnp.float32)
  p0_ref[...] = p[:, :C_PAD // 2]
  p1_ref[...] = p[:, C_PAD // 2:]
  r3_ref[...] = jnp.dot(h2, wr3_ref[...],
                        preferred_element_type=jnp.float32) + b3_ref[...]


def _layer23_tc(a0, a1, h0, h1, inv, wla, wlb, wra, wrb, b, wl3, wr3, b3):
  half = C_PAD // 2
  return pl.pallas_call(
      _layer23_body,
      grid=(N_PAD // BLK,),
      in_specs=[_row_spec(128), _row_spec(128), _row_spec(128), _row_spec(128),
                _row_spec(1),
                _full_spec(128, H), _full_spec(128, H),
                _full_spec(128, H), _full_spec(128, H), _full_spec(1, H),
                _full_spec(H, C_PAD), _full_spec(H, C_PAD),
                _full_spec(1, C_PAD)],
      out_specs=(_row_spec(half), _row_spec(half), _row_spec(C_PAD)),
      out_shape=(jax.ShapeDtypeStruct((N_PAD, half), jnp.float32),
                 jax.ShapeDtypeStruct((N_PAD, half), jnp.float32),
                 jax.ShapeDtypeStruct((N_PAD, C_PAD), jnp.float32)),
  )(a0, a1, h0, h1, inv, wla, wlb, wra, wrb, b, wl3, wr3, b3)


def _final_body(a0_ref, a1_ref, r_ref, inv_ref, out_ref):
  v = jnp.concatenate([a0_ref[...], a1_ref[...]], axis=1) * inv_ref[...]
  v = v + r_ref[...]
  col = lax.broadcasted_iota(jnp.int32, (BLK, C_PAD), 1)
  valid = col < C
  mx = jnp.max(jnp.where(valid, v, -jnp.inf), axis=1, keepdims=True)
  e = jnp.where(valid, jnp.exp(v - mx), 0.0)
  lse = jnp.log(jnp.sum(e, axis=1, keepdims=True))
  out_ref[...] = (v - mx - lse)[:, :C]


def _final_tc(a0, a1, r, inv):
  half = C_PAD // 2
  return pl.pallas_call(
      _final_body,
      grid=(N_PAD // BLK,),
      in_specs=[_row_spec(half), _row_spec(half), _row_spec(C_PAD),
                _row_spec(1)],
      out_specs=_row_spec(C),
      out_shape=jax.ShapeDtypeStruct((N_PAD, C), jnp.float32),
  )(a0, a1, r, inv)


def kernel(x, edge_index, Wl1, Wr1, b1, Wl2, Wr2, b2, Wl3, Wr3, b3):
  src = edge_index[0].astype(jnp.int32)
  dst = edge_index[1].astype(jnp.int32)
  src3 = src.reshape(NTILE, NCHUNK, CHUNK)
  dst3 = dst.reshape(NTILE, NCHUNK, CHUNK)

  xp = jnp.pad(x, ((0, N_PAD - N), (0, 0)))
  ones = jnp.ones((N_PAD, 16), jnp.float32)
  x0 = jnp.concatenate([xp[:, :64], ones], axis=1)
  x1 = jnp.concatenate([xp[:, 64:], ones], axis=1)
  z80 = jnp.zeros((ROWS_PER_TILE, 80), jnp.float32)
  a10, a11 = _agg80(x0, x1, src3, dst3, z80)

  h0, h1, inv = _layer1_tc(a10, a11, xp, Wl1[:64], Wl1[64:], Wr1,
                           b1.reshape(1, H))

  z128 = jnp.zeros((ROWS_PER_TILE, 128), jnp.float32)
  a20, a21 = _agg128(h0, h1, src3, dst3, z128)

  wl3 = jnp.pad(Wl3, ((0, 0), (0, C_PAD - C)))
  wr3 = jnp.pad(Wr3, ((0, 0), (0, C_PAD - C)))
  b3p = jnp.pad(b3, (0, C_PAD - C)).reshape(1, C_PAD)
  p0, p1, r3 = _layer23_tc(a20, a21, h0, h1, inv, Wl2[:128], Wl2[128:],
                           Wr2[:128], Wr2[128:], b2.reshape(1, H),
                           wl3, wr3, b3p)

  z32 = jnp.zeros((ROWS_PER_TILE, C_PAD // 2), jnp.float32)
  a30, a31 = _agg32(p0, p1, src3, dst3, z32)

  return _final_tc(a30, a31, r3, inv)[:N]
